# baseline (speedup 1.0000x reference)
import functools

import jax
import jax.numpy as jnp
from jax import lax
from jax.experimental import pallas as pl
from jax.experimental.pallas import tpu as pltpu
from jax.experimental.pallas import tpu_sc as plsc

_NC, _NS = 2, 16
_NW = _NC * _NS
_LANES = 16


def _make_sc_kernel(batch, in_dim, out_dim):
    rows_per_tile = batch // _NW
    pass_rows = 4
    npass = rows_per_tile // pass_rows
    w = 2048
    nchunk = out_dim // w
    nslice = out_dim // _NS

    mesh = plsc.VectorSubcoreMesh(
        core_axis_name="c", subcore_axis_name="s",
        num_cores=_NC, num_subcores=_NS)

    @functools.partial(
        pl.kernel,
        out_type=jax.ShapeDtypeStruct((batch, out_dim), jnp.float32),
        mesh=mesh,
        compiler_params=pltpu.CompilerParams(
            needs_layout_passes=False,
            disable_bounds_checks=True,
            disable_semaphore_checks=True,
        ),
        scratch_types=[
            pltpu.VMEM((pass_rows * in_dim,), jnp.float32),
            pltpu.VMEM((2 * 6 * w,), jnp.float32),
            pltpu.VMEM((2 * pass_rows * w,), jnp.float32),
            pltpu.VMEM_SHARED((6 * out_dim,), jnp.float32),
            pltpu.SemaphoreType.DMA,
            pltpu.SemaphoreType.DMA,
            pltpu.SemaphoreType.DMA,
            pltpu.SemaphoreType.DMA,
            pltpu.SemaphoreType.DMA,
            pltpu.SemaphoreType.DMA,
        ],
    )
    def sc_kernel(x_hbm, idx_hbm, w_hbm, out_hbm, xbuf, sbuf, obuf,
                  sh_comb, sem_x, sem_w, sem_ia, sem_ib, sem_oa, sem_ob):
        wbuf = sbuf
        cstage = obuf
        sem_i = [sem_ia, sem_ib]
        sem_o = [sem_oa, sem_ob]
        sid = lax.axis_index("s")
        wid = sid * _NC + lax.axis_index("c")
        rbase = wid * rows_per_tile

        def start_x(prow):
            return [pltpu.async_copy(x_hbm.at[prow + r],
                                     xbuf.at[pl.ds(r * in_dim, in_dim)],
                                     sem_x)
                    for r in range(pass_rows)]

        hx0 = start_x(rbase)

        hw = pltpu.async_copy(w_hbm.at[pl.ds(sid * nslice * 16, nslice * 16)],
                              wbuf.at[pl.ds(0, nslice * 16)], sem_w)

        @pl.when(sid == 0)
        def _():
            for c in range(nchunk):
                for half in range(2):
                    pltpu.async_copy(
                        idx_hbm.at[half, pl.ds(c * w, w)],
                        sh_comb.at[pl.ds(c * 6 * w + half * w, w)],
                        sem_ia)

        hw.wait()
        iota16 = lax.broadcasted_iota(jnp.int32, (_LANES,), 0) * 16

        @plsc.parallel_loop(0, nslice, step=_LANES, unroll=2)
        def cgroup(g):
            bv = g * 16 + iota16
            p = []
            for k in range(16):
                p.append(plsc.load_gather(wbuf, [bv + k]))
            m = p[0]
            for k in range(1, 16):
                m = jnp.maximum(m, p[k])
            p = [jnp.exp(v - m) for v in p]
            s = p[0]
            for k in range(1, 16):
                s = s + p[k]
            r = 1.0 / s
            p = [v * r for v in p]
            c0 = (((p[8] + p[9]) + (p[10] + p[11]))
                  + ((p[12] + p[13]) + (p[14] + p[15])))
            ca_ = (((p[2] + p[3]) + (p[6] + p[7]))
                   - ((p[8] + p[9]) + (p[12] + p[13])))
            cb_ = (((p[4] + p[5]) + (p[6] + p[7]))
                   - ((p[8] + p[9]) + (p[10] + p[11])))
            cab = (((p[1] - p[2]) - (p[4] + p[7]))
                   + ((p[8] + p[11]) + (p[13] - p[14]))
                   + 2.0 * (p[9] - p[6]))
            cstage[pl.ds(0 * nslice + g, _LANES)] = c0
            cstage[pl.ds(1 * nslice + g, _LANES)] = ca_
            cstage[pl.ds(2 * nslice + g, _LANES)] = cb_
            cstage[pl.ds(3 * nslice + g, _LANES)] = cab

        cslot = sid // 2
        coff = (sid % 2) * nslice
        for k in range(4):
            pltpu.sync_copy(
                cstage.at[pl.ds(k * nslice, nslice)],
                sh_comb.at[pl.ds(cslot * 6 * w + (2 + k) * w + coff,
                                 nslice)])

        @pl.when(sid == 0)
        def _():
            for c in range(nchunk):
                for half in range(2):
                    pltpu.make_async_copy(
                        idx_hbm.at[half, pl.ds(0, w)],
                        sh_comb.at[pl.ds(c * 6 * w + half * w, w)],
                        sem_ia).wait()

        plsc.subcore_barrier()

        def start_inputs(c, buf):
            pltpu.async_copy(sh_comb.at[pl.ds(c * 6 * w, 6 * w)],
                             sbuf.at[pl.ds(buf * 6 * w, 6 * w)], sem_i[buf])

        def wait_inputs(buf):
            pltpu.make_async_copy(sh_comb.at[pl.ds(0, 6 * w)],
                                  sbuf.at[pl.ds(buf * 6 * w, 6 * w)],
                                  sem_i[buf]).wait()

        def drain_outputs(prow, buf):
            for r in range(pass_rows):
                pltpu.make_async_copy(
                    obuf.at[pl.ds((pass_rows * buf + r) * w, w)],
                    out_hbm.at[prow + r, pl.ds(0, w)],
                    sem_o[buf]).wait()

        def compute_chunk(c, buf, prow):
            @plsc.parallel_loop(0, w, step=_LANES, unroll=4)
            def jbody(o):
                base = buf * 6 * w + o
                i0 = plsc.bitcast(sbuf[pl.ds(base, _LANES)], jnp.int32)
                i1 = plsc.bitcast(sbuf[pl.ds(base + w, _LANES)], jnp.int32)
                c0 = sbuf[pl.ds(base + 2 * w, _LANES)]
                ca = sbuf[pl.ds(base + 3 * w, _LANES)]
                cb = sbuf[pl.ds(base + 4 * w, _LANES)]
                cab = sbuf[pl.ds(base + 5 * w, _LANES)]
                for r in range(pass_rows):
                    a = plsc.load_gather(xbuf, [i0 + r * in_dim])
                    b = plsc.load_gather(xbuf, [i1 + r * in_dim])
                    obuf[pl.ds((pass_rows * buf + r) * w + o, _LANES)] = (
                        c0 + a * ca + b * cb + (a * b) * cab)

            for r in range(pass_rows):
                pltpu.async_copy(
                    obuf.at[pl.ds((pass_rows * buf + r) * w, w)],
                    out_hbm.at[prow + r, pl.ds(c * w, w)],
                    sem_o[buf])

        hx = hx0
        start_inputs(0, 0)
        for p in range(npass):
            prow = rbase + p * pass_rows
            for h in hx:
                h.wait()

            @pl.loop(0, nchunk // 2)
            def chunk_pair(t):
                c0_, c1_ = 2 * t, 2 * t + 1
                start_inputs(c1_, 1)
                wait_inputs(0)

                @pl.when(t > 0)
                def _():
                    drain_outputs(prow, 0)

                compute_chunk(c0_, 0, prow)

                @pl.when(t + 1 < nchunk // 2)
                def _():
                    start_inputs(c0_ + 2, 0)

                wait_inputs(1)

                @pl.when(t > 0)
                def _():
                    drain_outputs(prow, 1)

                compute_chunk(c1_, 1, prow)

            if p + 1 < npass:
                hx = start_x(rbase + (p + 1) * pass_rows)
                start_inputs(0, 0)
            drain_outputs(prow, 0)
            drain_outputs(prow, 1)

    return sc_kernel


def kernel(x, weight, indices):
    batch, in_dim = x.shape
    out_dim = weight.shape[0]
    idx = indices.astype(jnp.int32)
    idx_f = lax.bitcast_convert_type(idx, jnp.float32)
    w_flat = weight.reshape(-1)
    sc = _make_sc_kernel(batch, in_dim, out_dim)
    return sc(x, idx_f, w_flat)


# speedup vs baseline: 1.4216x; 1.4216x over previous
import functools

import jax
import jax.numpy as jnp
from jax import lax
from jax.experimental import pallas as pl
from jax.experimental.pallas import tpu as pltpu
from jax.experimental.pallas import tpu_sc as plsc

_NC, _NS = 2, 16
_NW = _NC * _NS
_LANES = 16


def _make_sc_kernel(batch, in_dim, out_dim):
    rows_per_tile = batch // _NW
    pass_rows = 4
    npass = rows_per_tile // pass_rows
    w = 2048
    nchunk = out_dim // w
    nslice = out_dim // _NS

    mesh = plsc.VectorSubcoreMesh(
        core_axis_name="c", subcore_axis_name="s",
        num_cores=_NC, num_subcores=_NS)

    @functools.partial(
        pl.kernel,
        out_type=jax.ShapeDtypeStruct((batch, out_dim), jnp.float32),
        mesh=mesh,
        compiler_params=pltpu.CompilerParams(
            needs_layout_passes=False,
            disable_bounds_checks=True,
            disable_semaphore_checks=True,
        ),
        scratch_types=[
            pltpu.VMEM((pass_rows * in_dim,), jnp.float32),
            pltpu.VMEM((2 * 2 * w,), jnp.int32),
            pltpu.VMEM((2 * 4 * w,), jnp.float32),
            pltpu.VMEM((2 * pass_rows * w,), jnp.float32),
            pltpu.VMEM_SHARED((2 * out_dim,), jnp.int32),
            pltpu.VMEM_SHARED((4 * out_dim,), jnp.float32),
            pltpu.SemaphoreType.DMA,
            pltpu.SemaphoreType.DMA,
            pltpu.SemaphoreType.DMA,
            pltpu.SemaphoreType.DMA,
            pltpu.SemaphoreType.DMA,
            pltpu.SemaphoreType.DMA,
        ],
    )
    def sc_kernel(x_hbm, idx_hbm, w_hbm, out_hbm, xbuf, ibuf, cbuf, obuf,
                  sh_idx, sh_coef,
                  sem_x, sem_w, sem_ia, sem_ib, sem_oa, sem_ob):
        wbuf = cbuf
        cstage = obuf
        sem_i = [sem_ia, sem_ib]
        sem_o = [sem_oa, sem_ob]
        sid = lax.axis_index("s")
        wid = sid * _NC + lax.axis_index("c")
        rbase = wid * rows_per_tile

        def start_x(prow):
            return [pltpu.async_copy(x_hbm.at[prow + r],
                                     xbuf.at[pl.ds(r * in_dim, in_dim)],
                                     sem_x)
                    for r in range(pass_rows)]

        hx0 = start_x(rbase)

        hw = pltpu.async_copy(w_hbm.at[pl.ds(sid * nslice * 16, nslice * 16)],
                              wbuf, sem_w)

        @pl.when(sid == 0)
        def _():
            pltpu.async_copy(idx_hbm.at[0], sh_idx.at[pl.ds(0, out_dim)],
                             sem_ia)
            pltpu.async_copy(idx_hbm.at[1], sh_idx.at[pl.ds(out_dim, out_dim)],
                             sem_ia)

        hw.wait()
        iota16 = lax.broadcasted_iota(jnp.int32, (_LANES,), 0) * 16

        @plsc.parallel_loop(0, nslice, step=_LANES, unroll=2)
        def cgroup(g):
            bv = g * 16 + iota16
            p = []
            for k in range(16):
                p.append(plsc.load_gather(wbuf, [bv + k]))
            m = p[0]
            for k in range(1, 16):
                m = jnp.maximum(m, p[k])
            p = [jnp.exp(v - m) for v in p]
            s = p[0]
            for k in range(1, 16):
                s = s + p[k]
            r = 1.0 / s
            p = [v * r for v in p]
            c0 = (((p[8] + p[9]) + (p[10] + p[11]))
                  + ((p[12] + p[13]) + (p[14] + p[15])))
            ca_ = (((p[2] + p[3]) + (p[6] + p[7]))
                   - ((p[8] + p[9]) + (p[12] + p[13])))
            cb_ = (((p[4] + p[5]) + (p[6] + p[7]))
                   - ((p[8] + p[9]) + (p[10] + p[11])))
            cab = (((p[1] - p[2]) - (p[4] + p[7]))
                   + ((p[8] + p[11]) + (p[13] - p[14]))
                   + 2.0 * (p[9] - p[6]))
            cstage[pl.ds(0 * nslice + g, _LANES)] = c0
            cstage[pl.ds(1 * nslice + g, _LANES)] = ca_
            cstage[pl.ds(2 * nslice + g, _LANES)] = cb_
            cstage[pl.ds(3 * nslice + g, _LANES)] = cab

        for k in range(4):
            pltpu.sync_copy(
                cstage.at[pl.ds(k * nslice, nslice)],
                sh_coef.at[pl.ds(k * out_dim + sid * nslice, nslice)])

        @pl.when(sid == 0)
        def _():
            pltpu.make_async_copy(idx_hbm.at[0],
                                  sh_idx.at[pl.ds(0, out_dim)],
                                  sem_ia).wait()
            pltpu.make_async_copy(idx_hbm.at[1],
                                  sh_idx.at[pl.ds(out_dim, out_dim)],
                                  sem_ia).wait()

        plsc.subcore_barrier()

        def start_inputs(c, buf):
            pltpu.async_copy(sh_idx.at[pl.ds(c * w, w)],
                             ibuf.at[pl.ds((2 * buf) * w, w)], sem_i[buf])
            pltpu.async_copy(sh_idx.at[pl.ds(out_dim + c * w, w)],
                             ibuf.at[pl.ds((2 * buf + 1) * w, w)], sem_i[buf])
            for k in range(4):
                pltpu.async_copy(sh_coef.at[pl.ds(k * out_dim + c * w, w)],
                                 cbuf.at[pl.ds((4 * buf + k) * w, w)],
                                 sem_i[buf])

        def wait_inputs(buf):
            pltpu.make_async_copy(sh_idx.at[pl.ds(0, w)],
                                  ibuf.at[pl.ds((2 * buf) * w, w)],
                                  sem_i[buf]).wait()
            pltpu.make_async_copy(sh_idx.at[pl.ds(0, w)],
                                  ibuf.at[pl.ds((2 * buf + 1) * w, w)],
                                  sem_i[buf]).wait()
            for k in range(4):
                pltpu.make_async_copy(sh_coef.at[pl.ds(0, w)],
                                      cbuf.at[pl.ds((4 * buf + k) * w, w)],
                                      sem_i[buf]).wait()

        def drain_outputs(prow, buf):
            for r in range(pass_rows):
                pltpu.make_async_copy(
                    obuf.at[pl.ds((pass_rows * buf + r) * w, w)],
                    out_hbm.at[prow + r, pl.ds(0, w)],
                    sem_o[buf]).wait()

        def compute_chunk(c, buf, prow):
            @plsc.parallel_loop(0, w, step=_LANES, unroll=4)
            def jbody(o):
                i0 = ibuf[pl.ds((2 * buf) * w + o, _LANES)]
                i1 = ibuf[pl.ds((2 * buf + 1) * w + o, _LANES)]
                c0 = cbuf[pl.ds((4 * buf) * w + o, _LANES)]
                ca = cbuf[pl.ds((4 * buf + 1) * w + o, _LANES)]
                cb = cbuf[pl.ds((4 * buf + 2) * w + o, _LANES)]
                cab = cbuf[pl.ds((4 * buf + 3) * w + o, _LANES)]
                for r in range(pass_rows):
                    a = plsc.load_gather(xbuf, [i0 + r * in_dim])
                    b = plsc.load_gather(xbuf, [i1 + r * in_dim])
                    obuf[pl.ds((pass_rows * buf + r) * w + o, _LANES)] = (
                        c0 + a * ca + b * cb + (a * b) * cab)

            for r in range(pass_rows):
                pltpu.async_copy(
                    obuf.at[pl.ds((pass_rows * buf + r) * w, w)],
                    out_hbm.at[prow + r, pl.ds(c * w, w)],
                    sem_o[buf])

        hx = hx0
        start_inputs(0, 0)
        for p in range(npass):
            prow = rbase + p * pass_rows
            for h in hx:
                h.wait()

            @pl.loop(0, nchunk // 2)
            def chunk_pair(t):
                c0_, c1_ = 2 * t, 2 * t + 1
                start_inputs(c1_, 1)
                wait_inputs(0)

                @pl.when(t > 0)
                def _():
                    drain_outputs(prow, 0)

                compute_chunk(c0_, 0, prow)

                @pl.when(t + 1 < nchunk // 2)
                def _():
                    start_inputs(c0_ + 2, 0)

                wait_inputs(1)

                @pl.when(t > 0)
                def _():
                    drain_outputs(prow, 1)

                compute_chunk(c1_, 1, prow)

            if p + 1 < npass:
                hx = start_x(rbase + (p + 1) * pass_rows)
                start_inputs(0, 0)
            drain_outputs(prow, 0)
            drain_outputs(prow, 1)

    return sc_kernel


def kernel(x, weight, indices):
    batch, in_dim = x.shape
    out_dim = weight.shape[0]
    idx = indices.astype(jnp.int32)
    w_flat = weight.reshape(-1)
    sc = _make_sc_kernel(batch, in_dim, out_dim)
    return sc(x, idx, w_flat)
